# Initial kernel scaffold; baseline (speedup 1.0000x reference)
import jax
import jax.numpy as jnp
from jax.experimental import pallas as pl


def kernel(targets_bbox, targets_labels, pred_offsets, pred_cls_logits, anchors, matches):
    raise NotImplementedError("write your pallas kernel here")


# speedup vs baseline: 6.3928x; 6.3928x over previous
import jax
import jax.numpy as jnp
from jax.experimental import pallas as pl
from jax.experimental.pallas import tpu as pltpu

_NEG_POS_RATIO = 3
_ALPHA = 1.0


def _phase1_body(tbT_ref, tl_ref, po_ref, lg_ref, an_ref, m_ref,
                 cls_ref, reg_ref):
    first = (pl.program_id(0) == 0) & (pl.program_id(1) == 0)
    A = lg_ref.shape[1]
    C = lg_ref.shape[2]
    G = tl_ref.shape[2]

    m = m_ref[0]
    fg = m >= 0
    safe = jnp.maximum(m, 0)
    g_iota = jax.lax.broadcasted_iota(jnp.int32, (A, G), 1)
    onehot = safe == g_iota

    tl = tl_ref[0]
    acls = jnp.sum(jnp.where(onehot, tl, 0), axis=1, keepdims=True)
    acls = jnp.where(fg, acls, 0)

    def gather_coord(j):
        row = tbT_ref[0, j, :].reshape(1, G)
        return jnp.sum(jnp.where(onehot, row, 0.0), axis=1, keepdims=True)

    gx0 = gather_coord(0)
    gy0 = gather_coord(1)
    gx1 = gather_coord(2)
    gy1 = gather_coord(3)

    an = an_ref[0]
    ax0 = an[:, 0:1]
    ay0 = an[:, 1:2]
    ax1 = an[:, 2:3]
    ay1 = an[:, 3:4]
    a_w = ax1 - ax0
    a_h = ay1 - ay0
    t0 = ((gx0 + gx1) - (ax0 + ax1)) * 0.5 / a_w
    t1 = ((gy0 + gy1) - (ay0 + ay1)) * 0.5 / a_h
    t2 = jnp.log((gx1 - gx0) / a_w)
    t3 = jnp.log((gy1 - gy0) / a_h)

    po = po_ref[0]
    sl1 = jnp.zeros_like(t0)
    for j, tj in enumerate((t0, t1, t2, t3)):
        d = jnp.abs(po[:, j:j + 1] - tj)
        sl1 = sl1 + jnp.where(d < 1.0, 0.5 * d * d, d - 0.5)
    regp = jnp.sum(jnp.where(fg, sl1, 0.0))

    lg = lg_ref[0]
    mx = jnp.max(lg, axis=1, keepdims=True)
    s = jnp.sum(jnp.exp(lg - mx), axis=1, keepdims=True)
    lse = mx + jnp.log(s)
    c_iota = jax.lax.broadcasted_iota(jnp.int32, (A, C), 1)
    picked = jnp.sum(jnp.where(c_iota == acls, lg, 0.0), axis=1,
                     keepdims=True)
    cls_ref[0] = lse - picked

    @pl.when(first)
    def _():
        reg_ref[...] = jnp.zeros((1, 1), jnp.float32)
    reg_ref[...] += regp.reshape(1, 1)


def _phase2_body(cls_ref, m_ref, reg_ref, out_ref):
    cls = cls_ref[...]
    m = m_ref[...]
    B = cls.shape[0]
    A = cls.shape[1]
    pos = m >= 0
    neg = jnp.logical_not(pos)
    posi = pos.astype(jnp.int32)
    num_pos = jnp.sum(posi, axis=1, keepdims=True)
    k = jnp.minimum(_NEG_POS_RATIO * num_pos, A - num_pos)

    clsc = jnp.maximum(cls, 0.0)
    bits = jax.lax.bitcast_convert_type(clsc, jnp.int32)
    bits = jnp.where(neg, bits, -1)

    def body(i, t):
        cand = t | jax.lax.shift_right_logical(jnp.int32(2 ** 30), i)
        cnt = jnp.sum((bits >= cand).astype(jnp.int32), axis=1,
                      keepdims=True)
        return jnp.where(cnt >= k, cand, t)

    t = jax.lax.fori_loop(0, 31, body, jnp.zeros((B, 1), jnp.int32))

    thr = jax.lax.bitcast_convert_type(t, jnp.float32)
    gt = neg & (bits > t)
    cnt_gt = jnp.sum(gt.astype(jnp.int32), axis=1, keepdims=True)
    sum_gt = jnp.sum(jnp.where(gt, clsc, 0.0), axis=1, keepdims=True)
    neg_sum = jnp.where(k > 0,
                        sum_gt + (k - cnt_gt).astype(jnp.float32) * thr,
                        0.0)
    pos_sum = jnp.sum(jnp.where(pos, clsc, 0.0), axis=1, keepdims=True)
    npt = jnp.maximum(1, jnp.sum(num_pos)).astype(jnp.float32)
    final_cls = (jnp.sum(pos_sum) + jnp.sum(neg_sum)) / npt
    out_ref[...] = _ALPHA * reg_ref[...] + final_cls.reshape(1, 1)


@jax.jit
def kernel(targets_bbox, targets_labels, pred_offsets, pred_cls_logits,
           anchors, matches):
    B, A, C = pred_cls_logits.shape
    G = targets_labels.shape[1]
    tbT = jnp.transpose(targets_bbox, (0, 2, 1))
    tl3 = targets_labels.reshape(B, 1, G).astype(jnp.int32)
    m3 = matches.reshape(B, A, 1).astype(jnp.int32)

    BLK = 1024
    cls, reg = pl.pallas_call(
        _phase1_body,
        grid=(B, A // BLK),
        in_specs=[
            pl.BlockSpec((1, 4, G), lambda b, j: (b, 0, 0)),
            pl.BlockSpec((1, 1, G), lambda b, j: (b, 0, 0)),
            pl.BlockSpec((1, BLK, 4), lambda b, j: (b, j, 0)),
            pl.BlockSpec((1, BLK, C), lambda b, j: (b, j, 0)),
            pl.BlockSpec((1, BLK, 4), lambda b, j: (b, j, 0)),
            pl.BlockSpec((1, BLK, 1), lambda b, j: (b, j, 0)),
        ],
        out_specs=[
            pl.BlockSpec((1, BLK, 1), lambda b, j: (b, j, 0)),
            pl.BlockSpec((1, 1), lambda b, j: (0, 0)),
        ],
        out_shape=[
            jax.ShapeDtypeStruct((B, A, 1), jnp.float32),
            jax.ShapeDtypeStruct((1, 1), jnp.float32),
        ],
    )(tbT, tl3, pred_offsets, pred_cls_logits, anchors, m3)

    out = pl.pallas_call(
        _phase2_body,
        out_shape=jax.ShapeDtypeStruct((1, 1), jnp.float32),
    )(cls.reshape(B, A), matches.astype(jnp.int32), reg)
    return out.reshape(())


# speedup vs baseline: 20.9781x; 3.2815x over previous
import jax
import jax.numpy as jnp
from jax.experimental import pallas as pl
from jax.experimental.pallas import tpu as pltpu

_NEG_POS_RATIO = 3
_ALPHA = 1.0


def _phase1_body(tbl_ref, lg_ref, anT_ref, poT_ref, ml_ref,
                 cls_ref, reg_ref):
    first = (pl.program_id(0) == 0) & (pl.program_id(1) == 0)
    RB = ml_ref.shape[1]
    L = ml_ref.shape[2]
    C = lg_ref.shape[3]
    G = tbl_ref.shape[2]

    m = ml_ref[0]
    fg = m >= 0
    safe = jnp.maximum(m, 0)

    zero = jnp.zeros((RB, L), jnp.float32)
    gx0 = zero
    gy0 = zero
    gx1 = zero
    gy1 = zero
    lab = zero
    for g in range(G):
        sel = safe == g
        gx0 = jnp.where(sel, tbl_ref[0, 0, g], gx0)
        gy0 = jnp.where(sel, tbl_ref[0, 1, g], gy0)
        gx1 = jnp.where(sel, tbl_ref[0, 2, g], gx1)
        gy1 = jnp.where(sel, tbl_ref[0, 3, g], gy1)
        lab = jnp.where(sel, tbl_ref[0, 4, g], lab)

    an = anT_ref[0]
    po = poT_ref[0]
    ax0 = an[0]
    ay0 = an[1]
    ax1 = an[2]
    ay1 = an[3]
    aw = ax1 - ax0
    ah = ay1 - ay0
    t0 = ((gx0 + gx1) - (ax0 + ax1)) * 0.5 / aw
    t1 = ((gy0 + gy1) - (ay0 + ay1)) * 0.5 / ah
    t2 = jnp.log((gx1 - gx0) / aw)
    t3 = jnp.log((gy1 - gy0) / ah)
    sl1 = zero
    for j, tj in enumerate((t0, t1, t2, t3)):
        d = jnp.abs(po[j] - tj)
        sl1 = sl1 + jnp.where(d < 1.0, 0.5 * d * d, d - 0.5)
    regp = jnp.sum(jnp.where(fg, sl1, 0.0))

    lg3 = lg_ref[0]
    mx = jnp.max(lg3, axis=2)
    e = jnp.exp(lg3 - mx[:, :, None])
    s = jnp.sum(e, axis=2)
    lse = mx + jnp.log(s)

    acls = jnp.where(fg, lab.astype(jnp.int32), 0)
    cio = jax.lax.broadcasted_iota(jnp.int32, (RB, L, C), 2)
    picked = jnp.sum(jnp.where(cio == acls[:, :, None], lg3, 0.0), axis=2)
    cls_ref[0] = lse - picked

    @pl.when(first)
    def _():
        reg_ref[...] = jnp.zeros((1, 1), jnp.float32)
    reg_ref[...] += regp.reshape(1, 1)


def _phase2_body(cls_ref, m_ref, reg_ref, out_ref):
    cls = cls_ref[...]
    m = m_ref[...]
    B = cls.shape[0]
    A = cls.shape[1]
    pos = m >= 0
    neg = jnp.logical_not(pos)
    posi = pos.astype(jnp.int32)
    num_pos = jnp.sum(posi, axis=1, keepdims=True)
    k = jnp.minimum(_NEG_POS_RATIO * num_pos, A - num_pos)

    clsc = jnp.maximum(cls, 0.0)
    bits = jax.lax.bitcast_convert_type(clsc, jnp.int32)
    bits = jnp.where(neg, bits, -1)

    def body(i, t):
        cand = t | jax.lax.shift_right_logical(jnp.int32(2 ** 30), i)
        cnt = jnp.sum((bits >= cand).astype(jnp.int32), axis=1,
                      keepdims=True)
        return jnp.where(cnt >= k, cand, t)

    t = jax.lax.fori_loop(0, 31, body, jnp.zeros((B, 1), jnp.int32))

    thr = jax.lax.bitcast_convert_type(t, jnp.float32)
    gt = neg & (bits > t)
    cnt_gt = jnp.sum(gt.astype(jnp.int32), axis=1, keepdims=True)
    sum_gt = jnp.sum(jnp.where(gt, clsc, 0.0), axis=1, keepdims=True)
    neg_sum = jnp.where(k > 0,
                        sum_gt + (k - cnt_gt).astype(jnp.float32) * thr,
                        0.0)
    pos_sum = jnp.sum(jnp.where(pos, clsc, 0.0), axis=1, keepdims=True)
    npt = jnp.maximum(1, jnp.sum(num_pos)).astype(jnp.float32)
    final_cls = (jnp.sum(pos_sum) + jnp.sum(neg_sum)) / npt
    out_ref[...] = _ALPHA * reg_ref[...] + final_cls.reshape(1, 1)


@jax.jit
def kernel(targets_bbox, targets_labels, pred_offsets, pred_cls_logits,
           anchors, matches):
    B, A, C = pred_cls_logits.shape
    G = targets_labels.shape[1]
    L = 128
    BLK = 1024
    RB = BLK // L

    tbl = jnp.concatenate(
        [jnp.transpose(targets_bbox, (0, 2, 1)),
         targets_labels.astype(jnp.float32)[:, None, :]], axis=1)
    lg4 = pred_cls_logits.reshape(B, A // L, L, C)
    anT = jnp.transpose(anchors, (0, 2, 1)).reshape(B, 4, A // L, L)
    poT = jnp.transpose(pred_offsets, (0, 2, 1)).reshape(B, 4, A // L, L)
    ml = matches.astype(jnp.int32).reshape(B, A // L, L)

    cls, reg = pl.pallas_call(
        _phase1_body,
        grid=(B, A // BLK),
        in_specs=[
            pl.BlockSpec((1, 5, G), lambda b, j: (b, 0, 0),
                         memory_space=pltpu.SMEM),
            pl.BlockSpec((1, RB, L, C), lambda b, j: (b, j, 0, 0)),
            pl.BlockSpec((1, 4, RB, L), lambda b, j: (b, 0, j, 0)),
            pl.BlockSpec((1, 4, RB, L), lambda b, j: (b, 0, j, 0)),
            pl.BlockSpec((1, RB, L), lambda b, j: (b, j, 0)),
        ],
        out_specs=[
            pl.BlockSpec((1, RB, L), lambda b, j: (b, j, 0)),
            pl.BlockSpec((1, 1), lambda b, j: (0, 0)),
        ],
        out_shape=[
            jax.ShapeDtypeStruct((B, A // L, L), jnp.float32),
            jax.ShapeDtypeStruct((1, 1), jnp.float32),
        ],
    )(tbl, lg4, anT, poT, ml)

    out = pl.pallas_call(
        _phase2_body,
        out_shape=jax.ShapeDtypeStruct((1, 1), jnp.float32),
    )(cls.reshape(B, A), matches.astype(jnp.int32), reg)
    return out.reshape(())


# speedup vs baseline: 32.3081x; 1.5401x over previous
import jax
import jax.numpy as jnp
from jax.experimental import pallas as pl
from jax.experimental.pallas import tpu as pltpu

_NEG_POS_RATIO = 3
_ALPHA = 1.0


def _phase1_body(tbl_ref, lg_ref, anT_ref, poT_ref, ml_ref,
                 cls_ref, reg_ref):
    first = (pl.program_id(0) == 0) & (pl.program_id(1) == 0)
    RB = ml_ref.shape[1]
    L = ml_ref.shape[2]
    C = lg_ref.shape[3]
    G = tbl_ref.shape[2]

    m = ml_ref[0]
    fg = m >= 0
    safe = jnp.maximum(m, 0)

    zero = jnp.zeros((RB, L), jnp.float32)
    gx0 = zero
    gy0 = zero
    gx1 = zero
    gy1 = zero
    lab = zero
    for g in range(G):
        sel = safe == g
        gx0 = jnp.where(sel, tbl_ref[0, 0, g], gx0)
        gy0 = jnp.where(sel, tbl_ref[0, 1, g], gy0)
        gx1 = jnp.where(sel, tbl_ref[0, 2, g], gx1)
        gy1 = jnp.where(sel, tbl_ref[0, 3, g], gy1)
        lab = jnp.where(sel, tbl_ref[0, 4, g], lab)

    an = anT_ref[0]
    po = poT_ref[0]
    ax0 = an[0]
    ay0 = an[1]
    ax1 = an[2]
    ay1 = an[3]
    aw = ax1 - ax0
    ah = ay1 - ay0
    t0 = ((gx0 + gx1) - (ax0 + ax1)) * 0.5 / aw
    t1 = ((gy0 + gy1) - (ay0 + ay1)) * 0.5 / ah
    t2 = jnp.log((gx1 - gx0) / aw)
    t3 = jnp.log((gy1 - gy0) / ah)
    sl1 = zero
    for j, tj in enumerate((t0, t1, t2, t3)):
        d = jnp.abs(po[j] - tj)
        sl1 = sl1 + jnp.where(d < 1.0, 0.5 * d * d, d - 0.5)
    regp = jnp.sum(jnp.where(fg, sl1, 0.0))

    lg3 = lg_ref[0]
    mx = jnp.max(lg3, axis=2)
    e = jnp.exp(lg3 - mx[:, :, None])
    s = jnp.sum(e, axis=2)
    lse = mx + jnp.log(s)

    acls = jnp.where(fg, lab.astype(jnp.int32), 0)
    cio = jax.lax.broadcasted_iota(jnp.int32, (RB, L, C), 2)
    picked = jnp.sum(jnp.where(cio == acls[:, :, None], lg3, 0.0), axis=2)
    cls_ref[0] = lse - picked

    @pl.when(first)
    def _():
        reg_ref[...] = jnp.zeros((1, 1), jnp.float32)
    reg_ref[...] += regp.reshape(1, 1)


def _phase2_body(cls_ref, m_ref, reg_ref, out_ref):
    cls = cls_ref[...]
    m = m_ref[...]
    B = cls.shape[0]
    A = cls.shape[1]
    pos = m >= 0
    neg = jnp.logical_not(pos)
    posi = pos.astype(jnp.int32)
    num_pos = jnp.sum(posi, axis=1, keepdims=True)
    k = jnp.minimum(_NEG_POS_RATIO * num_pos, A - num_pos)

    clsc = jnp.maximum(cls, 0.0)
    bits = jax.lax.bitcast_convert_type(clsc, jnp.int32)
    bits = jnp.where(neg, bits, -1)

    def body(i, t):
        cand = t | jax.lax.shift_right_logical(jnp.int32(2 ** 30), i)
        cnt = jnp.sum((bits >= cand).astype(jnp.int32), axis=1,
                      keepdims=True)
        return jnp.where(cnt >= k, cand, t)

    t = jax.lax.fori_loop(0, 31, body, jnp.zeros((B, 1), jnp.int32))

    thr = jax.lax.bitcast_convert_type(t, jnp.float32)
    gt = neg & (bits > t)
    cnt_gt = jnp.sum(gt.astype(jnp.int32), axis=1, keepdims=True)
    sum_gt = jnp.sum(jnp.where(gt, clsc, 0.0), axis=1, keepdims=True)
    neg_sum = jnp.where(k > 0,
                        sum_gt + (k - cnt_gt).astype(jnp.float32) * thr,
                        0.0)
    pos_sum = jnp.sum(jnp.where(pos, clsc, 0.0), axis=1, keepdims=True)
    npt = jnp.maximum(1, jnp.sum(num_pos)).astype(jnp.float32)
    final_cls = (jnp.sum(pos_sum) + jnp.sum(neg_sum)) / npt
    out_ref[...] = _ALPHA * reg_ref[...] + final_cls.reshape(1, 1)


@jax.jit
def kernel(targets_bbox, targets_labels, pred_offsets, pred_cls_logits,
           anchors, matches):
    B, A, C = pred_cls_logits.shape
    G = targets_labels.shape[1]
    L = 128
    BLK = 8192
    RB = BLK // L

    tbl = jnp.concatenate(
        [jnp.transpose(targets_bbox, (0, 2, 1)),
         targets_labels.astype(jnp.float32)[:, None, :]], axis=1)
    lg4 = pred_cls_logits.reshape(B, A // L, L, C)
    anT = jnp.transpose(anchors, (0, 2, 1)).reshape(B, 4, A // L, L)
    poT = jnp.transpose(pred_offsets, (0, 2, 1)).reshape(B, 4, A // L, L)
    ml = matches.astype(jnp.int32).reshape(B, A // L, L)

    cls, reg = pl.pallas_call(
        _phase1_body,
        grid=(B, A // BLK),
        in_specs=[
            pl.BlockSpec((1, 5, G), lambda b, j: (b, 0, 0),
                         memory_space=pltpu.SMEM),
            pl.BlockSpec((1, RB, L, C), lambda b, j: (b, j, 0, 0)),
            pl.BlockSpec((1, 4, RB, L), lambda b, j: (b, 0, j, 0)),
            pl.BlockSpec((1, 4, RB, L), lambda b, j: (b, 0, j, 0)),
            pl.BlockSpec((1, RB, L), lambda b, j: (b, j, 0)),
        ],
        out_specs=[
            pl.BlockSpec((1, RB, L), lambda b, j: (b, j, 0)),
            pl.BlockSpec((1, 1), lambda b, j: (0, 0)),
        ],
        out_shape=[
            jax.ShapeDtypeStruct((B, A // L, L), jnp.float32),
            jax.ShapeDtypeStruct((1, 1), jnp.float32),
        ],
    )(tbl, lg4, anT, poT, ml)

    out = pl.pallas_call(
        _phase2_body,
        out_shape=jax.ShapeDtypeStruct((1, 1), jnp.float32),
    )(cls.reshape(B, A), matches.astype(jnp.int32), reg)
    return out.reshape(())
